# Initial kernel scaffold; baseline (speedup 1.0000x reference)
import jax
import jax.numpy as jnp
from jax.experimental import pallas as pl


def kernel(x, s1, t1, w1, s2, t2, w2, s3, t3, w3, s4, t4, w4, f1_w, f1_b, f2_w, f2_b, f3_w, f3_b):
    raise NotImplementedError("write your pallas kernel here")


# speedup vs baseline: 27.0386x; 27.0386x over previous
import jax
import jax.numpy as jnp
from jax import lax
from jax.experimental import pallas as pl
from jax.experimental.pallas import tpu as pltpu

G = 8
WP2 = 64
NP2 = 43 * WP2
M1 = 40 * WP2
WP3 = 32
NP3 = 23 * WP3
M3 = 20 * WP3
WP4 = 16
NP4 = 13 * WP4
M4 = 10 * WP4


def _net_kernel(xg_ref,
                w1_ref, s1_ref, t1_ref,
                w2_ref, s2_ref, t2_ref,
                w3_ref, s3_ref, t3_ref,
                w4_ref, s4_ref, t4_ref,
                se2_ref, so2_ref, se3_ref, so3_ref, se4_ref, so4_ref,
                perm_ref,
                f1_ref, b1_ref, f2_ref, b2_ref, f3_ref, b3_ref,
                o_ref,
                p2, cs2, p3, cs3, p4, cs4, a4pi, a4):
    f32 = jnp.float32

    acc1 = jnp.zeros((M1, G * 8), f32)
    for t in range(9):
        off = (t // 3) * WP2 + (t % 3)
        acc1 = acc1 + jnp.dot(xg_ref[0, pl.ds(off, M1), :], w1_ref[t],
                              preferred_element_type=f32)
    y1 = jnp.maximum(acc1 * s1_ref[...] + t1_ref[...], 0.0)
    rowmask = (lax.broadcasted_iota(jnp.int32, (M1, G * 8), 0) & (WP2 - 1)) < 40
    y1 = jnp.where(rowmask, y1, 0.0)
    p2[pl.ds(0, 72), :] = jnp.zeros((72, G * 8), f32)
    p2[pl.ds(NP2 - 128, 128), :] = jnp.zeros((128, G * 8), f32)
    p2[pl.ds(WP2 + 1, M1), :] = y1

    acc = jnp.zeros((M1, G * 16), f32)
    for t in range(9):
        off = (t // 3) * WP2 + (t % 3)
        acc = acc + jnp.dot(p2[pl.ds(off, M1), :], w2_ref[t],
                            preferred_element_type=f32)
    cs2[...] = jnp.maximum(acc * s2_ref[...] + t2_ref[...], 0.0)

    p3[...] = jnp.zeros_like(p3)
    for hp in range(20):
        r0 = (2 * hp) * WP2
        mv = jnp.maximum(cs2[pl.ds(r0, WP2), :], cs2[pl.ds(r0 + WP2, WP2), :])
        m = jnp.maximum(jnp.dot(se2_ref[...], mv, preferred_element_type=f32),
                        jnp.dot(so2_ref[...], mv, preferred_element_type=f32))
        p3[pl.ds((hp + 1) * WP3 + 1, 20), :] = m

    acc3 = jnp.zeros((M3, G * 32), f32)
    for t in range(9):
        off = (t // 3) * WP3 + (t % 3)
        acc3 = acc3 + jnp.dot(p3[pl.ds(off, M3), :], w3_ref[t],
                              preferred_element_type=f32)
    cs3[...] = jnp.maximum(acc3 * s3_ref[...] + t3_ref[...], 0.0)

    p4[...] = jnp.zeros_like(p4)
    for hp in range(10):
        r0 = (2 * hp) * WP3
        mv = jnp.maximum(cs3[pl.ds(r0, WP3), :], cs3[pl.ds(r0 + WP3, WP3), :])
        m = jnp.maximum(jnp.dot(se3_ref[...], mv, preferred_element_type=f32),
                        jnp.dot(so3_ref[...], mv, preferred_element_type=f32))
        p4[pl.ds((hp + 1) * WP4 + 1, 10), :] = m

    acc4a = jnp.zeros((M4, 256), f32)
    acc4b = jnp.zeros((M4, 256), f32)
    for t in range(9):
        off = (t // 3) * WP4 + (t % 3)
        acc4a = acc4a + jnp.dot(p4[pl.ds(off, M4), 0:128], w4_ref[t],
                                preferred_element_type=f32)
        acc4b = acc4b + jnp.dot(p4[pl.ds(off, M4), 128:256], w4_ref[t],
                                preferred_element_type=f32)
    cs4[:, 0:256] = jnp.maximum(acc4a * s4_ref[...] + t4_ref[...], 0.0)
    cs4[:, 256:512] = jnp.maximum(acc4b * s4_ref[...] + t4_ref[...], 0.0)

    for hp in range(5):
        r0 = (2 * hp) * WP4
        mv = jnp.maximum(cs4[pl.ds(r0, WP4), :], cs4[pl.ds(r0 + WP4, WP4), :])
        m = jnp.maximum(jnp.dot(se4_ref[...], mv, preferred_element_type=f32),
                        jnp.dot(so4_ref[...], mv, preferred_element_type=f32))
        for g in range(G):
            a4pi[pl.ds(g * 25 + 5 * hp, 5), :] = m[:, 64 * g:64 * (g + 1)]

    a4[...] = jnp.dot(perm_ref[...], a4pi[...], preferred_element_type=f32)

    h = jnp.zeros((G, 256), f32) + b1_ref[...]
    for r in range(25):
        h = h + jnp.dot(a4[pl.ds(8 * r, 8), :], f1_ref[pl.ds(64 * r, 64), :],
                        preferred_element_type=f32)
    h = jnp.maximum(h, 0.0)
    h2 = jnp.maximum(jnp.dot(h, f2_ref[...], preferred_element_type=f32)
                     + b2_ref[...], 0.0)
    o_ref[...] = jnp.dot(h2, f3_ref[...], preferred_element_type=f32) \
        + b3_ref[...]


def _evenodd(n_out, n_in):
    e = jnp.zeros((n_out, n_in), jnp.float32)
    idx = jnp.arange(n_out)
    e = e.at[idx, 2 * idx].set(1.0)
    o = jnp.zeros((n_out, n_in), jnp.float32)
    o = o.at[idx, 2 * idx + 1].set(1.0)
    return e, o


def kernel(x, s1, t1, w1, s2, t2, w2, s3, t3, w3, s4, t4, w4,
           f1_w, f1_b, f2_w, f2_b, f3_w, f3_b):
    B = x.shape[0]
    nsteps = B // G
    f32 = jnp.float32

    x2 = x[:, 0]
    xp = jnp.pad(x2, ((0, 0), (1, 1), (1, WP2 - 1 - 40)))
    xp = jnp.pad(xp.reshape(B, 42 * WP2), ((0, 0), (0, NP2 - 42 * WP2)))
    xpt = xp.reshape(nsteps, G, NP2).transpose(0, 2, 1)

    eyeG = jnp.eye(G, dtype=f32)
    w1g = (eyeG[None, :, :, None] * w1[:, None, None, :]).reshape(9, G, G * 8)
    kron8 = lambda m: jnp.kron(eyeG, m)
    w2g = jax.vmap(kron8)(w2.reshape(9, 8, 16))
    w3g = jax.vmap(kron8)(w3.reshape(9, 16, 32))
    eye4 = jnp.eye(4, dtype=f32)
    w4g = jax.vmap(lambda m: jnp.kron(eye4, m))(w4.reshape(9, 32, 64))
    s1g, t1g = jnp.tile(s1, (1, G)), jnp.tile(t1, (1, G))
    s2g, t2g = jnp.tile(s2, (1, G)), jnp.tile(t2, (1, G))
    s3g, t3g = jnp.tile(s3, (1, G)), jnp.tile(t3, (1, G))
    s4g, t4g = jnp.tile(s4, (1, 4)), jnp.tile(t4, (1, 4))
    se2, so2 = _evenodd(20, WP2)
    se3, so3 = _evenodd(10, WP3)
    se4, so4 = _evenodd(5, WP4)
    ridx = jnp.arange(25 * G)
    perm = jnp.zeros((25 * G, 25 * G), f32).at[
        ridx, (ridx % G) * 25 + ridx // G].set(1.0)

    full = lambda shape: pl.BlockSpec(shape, lambda b: tuple(0 for _ in shape))

    grid_spec = pltpu.PrefetchScalarGridSpec(
        num_scalar_prefetch=0,
        grid=(nsteps,),
        in_specs=[
            pl.BlockSpec((1, NP2, G), lambda b: (b, 0, 0)),
            full((9, G, G * 8)), full((1, G * 8)), full((1, G * 8)),
            full((9, 64, 128)), full((1, 128)), full((1, 128)),
            full((9, 128, 256)), full((1, 256)), full((1, 256)),
            full((9, 128, 256)), full((1, 256)), full((1, 256)),
            full((20, WP2)), full((20, WP2)),
            full((10, WP3)), full((10, WP3)),
            full((5, WP4)), full((5, WP4)),
            full((25 * G, 25 * G)),
            full((1600, 256)), full((1, 256)),
            full((256, 16)), full((1, 16)),
            full((16, 2)), full((1, 2)),
        ],
        out_specs=pl.BlockSpec((G, 2), lambda b: (b, 0)),
        scratch_shapes=[
            pltpu.VMEM((NP2, G * 8), f32),
            pltpu.VMEM((M1, G * 16), f32),
            pltpu.VMEM((NP3, G * 16), f32),
            pltpu.VMEM((M3, G * 32), f32),
            pltpu.VMEM((NP4, G * 32), f32),
            pltpu.VMEM((M4, G * 64), f32),
            pltpu.VMEM((25 * G, 64), f32),
            pltpu.VMEM((25 * G, 64), f32),
        ],
    )
    out = pl.pallas_call(
        _net_kernel,
        out_shape=jax.ShapeDtypeStruct((B, 2), f32),
        grid_spec=grid_spec,
        compiler_params=pltpu.CompilerParams(
            dimension_semantics=("parallel",)),
    )(xpt,
      w1g, s1g, t1g, w2g, s2g, t2g, w3g, s3g, t3g, w4g, s4g, t4g,
      se2, so2, se3, so3, se4, so4, perm,
      f1_w, f1_b, f2_w, f2_b, f3_w, f3_b)
    return out


# speedup vs baseline: 43.0853x; 1.5935x over previous
import jax
import jax.numpy as jnp
from jax import lax
from jax.experimental import pallas as pl
from jax.experimental.pallas import tpu as pltpu

G = 16
WP2 = 64
NP2 = 43 * WP2
M1 = 40 * WP2
WP3 = 32
NP3 = 23 * WP3
M3 = 20 * WP3
WP4 = 16
NP4 = 13 * WP4
M4 = 10 * WP4


def _net_kernel(xg_ref,
                w1_ref, s1_ref, t1_ref,
                w2_ref, s2_ref, t2_ref,
                w3_ref, s3_ref, t3_ref,
                w4_ref, s4_ref, t4_ref,
                se2_ref, so2_ref, se3_ref, so3_ref, se4_ref, so4_ref,
                perm_ref,
                f1_ref, b1_ref, f2_ref, b2_ref, f3_ref, b3_ref,
                o_ref,
                p2, cs2, p3, cs3, p4, cs4, a4pi, a4):
    f32 = jnp.float32

    acc1a = jnp.zeros((M1, G * 8), f32)
    acc1b = jnp.zeros((M1, G * 8), f32)
    for t in range(9):
        off = (t // 3) * WP2 + (t % 3)
        d = jnp.dot(xg_ref[0, pl.ds(off, M1), :], w1_ref[t],
                    preferred_element_type=f32)
        if t % 2 == 0:
            acc1a = acc1a + d
        else:
            acc1b = acc1b + d
    y1 = jnp.maximum((acc1a + acc1b) * s1_ref[...] + t1_ref[...], 0.0)
    rowmask = (lax.broadcasted_iota(jnp.int32, (M1, G * 8), 0) & (WP2 - 1)) < 40
    y1 = jnp.where(rowmask, y1, 0.0)
    p2[pl.ds(0, 72), :] = jnp.zeros((72, G * 8), f32)
    p2[pl.ds(NP2 - 128, 128), :] = jnp.zeros((128, G * 8), f32)
    p2[pl.ds(WP2 + 1, M1), :] = y1

    for u in range(2):
        acc2 = jnp.zeros((M1, 128), f32)
        for t in range(9):
            off = (t // 3) * WP2 + (t % 3)
            acc2 = acc2 + jnp.dot(p2[pl.ds(off, M1), 64 * u:64 * (u + 1)],
                                  w2_ref[t], preferred_element_type=f32)
        cs2[:, 128 * u:128 * (u + 1)] = jnp.maximum(
            acc2 * s2_ref[...] + t2_ref[...], 0.0)

    p3[...] = jnp.zeros_like(p3)
    for hp in range(20):
        r0 = (2 * hp) * WP2
        mv = jnp.maximum(cs2[pl.ds(r0, WP2), :], cs2[pl.ds(r0 + WP2, WP2), :])
        m = jnp.maximum(jnp.dot(se2_ref[...], mv, preferred_element_type=f32),
                        jnp.dot(so2_ref[...], mv, preferred_element_type=f32))
        p3[pl.ds((hp + 1) * WP3 + 1, 20), :] = m

    for q in range(4):
        acc3 = jnp.zeros((M3, 128), f32)
        for t in range(9):
            off = (t // 3) * WP3 + (t % 3)
            acc3 = acc3 + jnp.dot(p3[pl.ds(off, M3), 64 * q:64 * (q + 1)],
                                  w3_ref[t], preferred_element_type=f32)
        cs3[:, 128 * q:128 * (q + 1)] = jnp.maximum(
            acc3 * s3_ref[...] + t3_ref[...], 0.0)

    p4[...] = jnp.zeros_like(p4)
    for hp in range(10):
        r0 = (2 * hp) * WP3
        mv = jnp.maximum(cs3[pl.ds(r0, WP3), :], cs3[pl.ds(r0 + WP3, WP3), :])
        m = jnp.maximum(jnp.dot(se3_ref[...], mv, preferred_element_type=f32),
                        jnp.dot(so3_ref[...], mv, preferred_element_type=f32))
        p4[pl.ds((hp + 1) * WP4 + 1, 10), :] = m

    for e in range(8):
        acc4 = jnp.zeros((M4, 128), f32)
        for t in range(9):
            off = (t // 3) * WP4 + (t % 3)
            acc4 = acc4 + jnp.dot(p4[pl.ds(off, M4), 64 * e:64 * (e + 1)],
                                  w4_ref[t], preferred_element_type=f32)
        cs4[:, 128 * e:128 * (e + 1)] = jnp.maximum(
            acc4 * s4_ref[...] + t4_ref[...], 0.0)

    for hp in range(5):
        r0 = (2 * hp) * WP4
        mv = jnp.maximum(cs4[pl.ds(r0, WP4), :], cs4[pl.ds(r0 + WP4, WP4), :])
        m = jnp.maximum(jnp.dot(se4_ref[...], mv, preferred_element_type=f32),
                        jnp.dot(so4_ref[...], mv, preferred_element_type=f32))
        for g in range(G):
            a4pi[pl.ds(g * 25 + 5 * hp, 5), :] = m[:, 64 * g:64 * (g + 1)]

    a4[...] = jnp.dot(perm_ref[...], a4pi[...], preferred_element_type=f32)

    ha = jnp.zeros((G, 256), f32) + b1_ref[...]
    hb = jnp.zeros((G, 256), f32)
    for r in range(25):
        d = jnp.dot(a4[pl.ds(G * r, G), :], f1_ref[pl.ds(64 * r, 64), :],
                    preferred_element_type=f32)
        if r % 2 == 0:
            ha = ha + d
        else:
            hb = hb + d
    h = jnp.maximum(ha + hb, 0.0)
    h2 = jnp.maximum(jnp.dot(h, f2_ref[...], preferred_element_type=f32)
                     + b2_ref[...], 0.0)
    o_ref[...] = jnp.dot(h2, f3_ref[...], preferred_element_type=f32) \
        + b3_ref[...]


def _evenodd(n_out, n_in):
    e = jnp.zeros((n_out, n_in), jnp.float32)
    idx = jnp.arange(n_out)
    e = e.at[idx, 2 * idx].set(1.0)
    o = jnp.zeros((n_out, n_in), jnp.float32)
    o = o.at[idx, 2 * idx + 1].set(1.0)
    return e, o


def kernel(x, s1, t1, w1, s2, t2, w2, s3, t3, w3, s4, t4, w4,
           f1_w, f1_b, f2_w, f2_b, f3_w, f3_b):
    B = x.shape[0]
    nsteps = B // G
    f32 = jnp.float32

    x2 = x[:, 0]
    xp = jnp.pad(x2, ((0, 0), (1, 1), (1, WP2 - 1 - 40)))
    xp = jnp.pad(xp.reshape(B, 42 * WP2), ((0, 0), (0, NP2 - 42 * WP2)))
    xpt = xp.reshape(nsteps, G, NP2).transpose(0, 2, 1)

    eyeG = jnp.eye(G, dtype=f32)
    w1g = (eyeG[None, :, :, None] * w1[:, None, None, :]).reshape(9, G, G * 8)
    eye8 = jnp.eye(8, dtype=f32)
    w2g = jax.vmap(lambda m: jnp.kron(eye8, m))(w2.reshape(9, 8, 16))
    eye4 = jnp.eye(4, dtype=f32)
    w3g = jax.vmap(lambda m: jnp.kron(eye4, m))(w3.reshape(9, 16, 32))
    eye2 = jnp.eye(2, dtype=f32)
    w4g = jax.vmap(lambda m: jnp.kron(eye2, m))(w4.reshape(9, 32, 64))
    s1g, t1g = jnp.tile(s1, (1, G)), jnp.tile(t1, (1, G))
    s2g, t2g = jnp.tile(s2, (1, 8)), jnp.tile(t2, (1, 8))
    s3g, t3g = jnp.tile(s3, (1, 4)), jnp.tile(t3, (1, 4))
    s4g, t4g = jnp.tile(s4, (1, 2)), jnp.tile(t4, (1, 2))
    se2, so2 = _evenodd(20, WP2)
    se3, so3 = _evenodd(10, WP3)
    se4, so4 = _evenodd(5, WP4)
    ridx = jnp.arange(25 * G)
    perm = jnp.zeros((25 * G, 25 * G), f32).at[
        ridx, (ridx % G) * 25 + ridx // G].set(1.0)

    full = lambda shape: pl.BlockSpec(shape, lambda b: tuple(0 for _ in shape))

    grid_spec = pltpu.PrefetchScalarGridSpec(
        num_scalar_prefetch=0,
        grid=(nsteps,),
        in_specs=[
            pl.BlockSpec((1, NP2, G), lambda b: (b, 0, 0)),
            full((9, G, G * 8)), full((1, G * 8)), full((1, G * 8)),
            full((9, 64, 128)), full((1, 128)), full((1, 128)),
            full((9, 64, 128)), full((1, 128)), full((1, 128)),
            full((9, 64, 128)), full((1, 128)), full((1, 128)),
            full((20, WP2)), full((20, WP2)),
            full((10, WP3)), full((10, WP3)),
            full((5, WP4)), full((5, WP4)),
            full((25 * G, 25 * G)),
            full((1600, 256)), full((1, 256)),
            full((256, 16)), full((1, 16)),
            full((16, 2)), full((1, 2)),
        ],
        out_specs=pl.BlockSpec((G, 2), lambda b: (b, 0)),
        scratch_shapes=[
            pltpu.VMEM((NP2, G * 8), f32),
            pltpu.VMEM((M1, G * 16), f32),
            pltpu.VMEM((NP3, G * 16), f32),
            pltpu.VMEM((M3, G * 32), f32),
            pltpu.VMEM((NP4, G * 32), f32),
            pltpu.VMEM((M4, G * 64), f32),
            pltpu.VMEM((25 * G, 64), f32),
            pltpu.VMEM((25 * G, 64), f32),
        ],
    )
    out = pl.pallas_call(
        _net_kernel,
        out_shape=jax.ShapeDtypeStruct((B, 2), f32),
        grid_spec=grid_spec,
        compiler_params=pltpu.CompilerParams(
            dimension_semantics=("parallel",)),
    )(xpt,
      w1g, s1g, t1g, w2g, s2g, t2g, w3g, s3g, t3g, w4g, s4g, t4g,
      se2, so2, se3, so3, se4, so4, perm,
      f1_w, f1_b, f2_w, f2_b, f3_w, f3_b)
    return out


# speedup vs baseline: 60.1357x; 1.3957x over previous
import jax
import jax.numpy as jnp
from jax import lax
from jax.experimental import pallas as pl
from jax.experimental.pallas import tpu as pltpu

G = 32
WP2 = 64
NP2 = 43 * WP2
M1 = 40 * WP2
WP3 = 32
NP3 = 23 * WP3
M3 = 20 * WP3
WP4 = 16
NP4 = 13 * WP4
M4 = 10 * WP4


def _net_kernel(xg_ref,
                w1_ref, s1_ref, t1_ref,
                w2_ref, s2_ref, t2_ref,
                w3_ref, s3_ref, t3_ref,
                w4_ref, s4_ref, t4_ref,
                se2_ref, so2_ref, se3_ref, so3_ref, se4_ref, so4_ref,
                perm_ref,
                f1_ref, b1_ref, f2_ref, b2_ref, f3_ref, b3_ref,
                o_ref,
                p2, cs2, p3, cs3, p4, cs4, a4pi, a4):
    f32 = jnp.float32
    rowmask = (lax.broadcasted_iota(jnp.int32, (M1, 128), 0) & (WP2 - 1)) < 40

    p2[pl.ds(0, 72), :] = jnp.zeros((72, G * 8), f32)
    p2[pl.ds(NP2 - 128, 128), :] = jnp.zeros((128, G * 8), f32)
    for v in range(G // 16):
        acc1 = jnp.zeros((M1, 128), f32)
        for t in range(9):
            off = (t // 3) * WP2 + (t % 3)
            acc1 = acc1 + jnp.dot(xg_ref[0, pl.ds(off, M1),
                                         16 * v:16 * (v + 1)],
                                  w1_ref[t], preferred_element_type=f32)
        y1 = jnp.maximum(acc1 * s1_ref[...] + t1_ref[...], 0.0)
        y1 = jnp.where(rowmask, y1, 0.0)
        p2[pl.ds(WP2 + 1, M1), 128 * v:128 * (v + 1)] = y1

    for u in range(G // 8):
        acc2 = jnp.zeros((M1, 128), f32)
        for t in range(9):
            off = (t // 3) * WP2 + (t % 3)
            acc2 = acc2 + jnp.dot(p2[pl.ds(off, M1), 64 * u:64 * (u + 1)],
                                  w2_ref[t], preferred_element_type=f32)
        cs2[:, 128 * u:128 * (u + 1)] = jnp.maximum(
            acc2 * s2_ref[...] + t2_ref[...], 0.0)

    p3[...] = jnp.zeros_like(p3)
    for hp in range(20):
        r0 = (2 * hp) * WP2
        mv = jnp.maximum(cs2[pl.ds(r0, WP2), :], cs2[pl.ds(r0 + WP2, WP2), :])
        m = jnp.maximum(jnp.dot(se2_ref[...], mv, preferred_element_type=f32),
                        jnp.dot(so2_ref[...], mv, preferred_element_type=f32))
        p3[pl.ds((hp + 1) * WP3 + 1, 20), :] = m

    for q in range(G // 4):
        acc3 = jnp.zeros((M3, 128), f32)
        for t in range(9):
            off = (t // 3) * WP3 + (t % 3)
            acc3 = acc3 + jnp.dot(p3[pl.ds(off, M3), 64 * q:64 * (q + 1)],
                                  w3_ref[t], preferred_element_type=f32)
        cs3[:, 128 * q:128 * (q + 1)] = jnp.maximum(
            acc3 * s3_ref[...] + t3_ref[...], 0.0)

    p4[...] = jnp.zeros_like(p4)
    for hp in range(10):
        r0 = (2 * hp) * WP3
        mv = jnp.maximum(cs3[pl.ds(r0, WP3), :], cs3[pl.ds(r0 + WP3, WP3), :])
        m = jnp.maximum(jnp.dot(se3_ref[...], mv, preferred_element_type=f32),
                        jnp.dot(so3_ref[...], mv, preferred_element_type=f32))
        p4[pl.ds((hp + 1) * WP4 + 1, 10), :] = m

    for e in range(G // 2):
        acc4 = jnp.zeros((M4, 128), f32)
        for t in range(9):
            off = (t // 3) * WP4 + (t % 3)
            acc4 = acc4 + jnp.dot(p4[pl.ds(off, M4), 64 * e:64 * (e + 1)],
                                  w4_ref[t], preferred_element_type=f32)
        cs4[:, 128 * e:128 * (e + 1)] = jnp.maximum(
            acc4 * s4_ref[...] + t4_ref[...], 0.0)

    for hp in range(5):
        r0 = (2 * hp) * WP4
        mv = jnp.maximum(cs4[pl.ds(r0, WP4), :], cs4[pl.ds(r0 + WP4, WP4), :])
        m = jnp.maximum(jnp.dot(se4_ref[...], mv, preferred_element_type=f32),
                        jnp.dot(so4_ref[...], mv, preferred_element_type=f32))
        for g in range(G):
            a4pi[pl.ds(g * 25 + 5 * hp, 5), :] = m[:, 64 * g:64 * (g + 1)]

    for v in range(G // 16):
        a4[...] = jnp.dot(perm_ref[...], a4pi[pl.ds(400 * v, 400), :],
                          preferred_element_type=f32)
        ha = jnp.zeros((16, 256), f32) + b1_ref[...]
        hb = jnp.zeros((16, 256), f32)
        for r in range(25):
            d = jnp.dot(a4[pl.ds(16 * r, 16), :], f1_ref[pl.ds(64 * r, 64), :],
                        preferred_element_type=f32)
            if r % 2 == 0:
                ha = ha + d
            else:
                hb = hb + d
        h = jnp.maximum(ha + hb, 0.0)
        h2 = jnp.maximum(jnp.dot(h, f2_ref[...], preferred_element_type=f32)
                         + b2_ref[...], 0.0)
        o_ref[pl.ds(16 * v, 16), :] = jnp.dot(
            h2, f3_ref[...], preferred_element_type=f32) + b3_ref[...]


def _evenodd(n_out, n_in):
    e = jnp.zeros((n_out, n_in), jnp.float32)
    idx = jnp.arange(n_out)
    e = e.at[idx, 2 * idx].set(1.0)
    o = jnp.zeros((n_out, n_in), jnp.float32)
    o = o.at[idx, 2 * idx + 1].set(1.0)
    return e, o


def kernel(x, s1, t1, w1, s2, t2, w2, s3, t3, w3, s4, t4, w4,
           f1_w, f1_b, f2_w, f2_b, f3_w, f3_b):
    B = x.shape[0]
    nsteps = B // G
    f32 = jnp.float32

    x2 = x[:, 0]
    xp = jnp.pad(x2, ((0, 0), (1, 1), (1, WP2 - 1 - 40)))
    xp = jnp.pad(xp.reshape(B, 42 * WP2), ((0, 0), (0, NP2 - 42 * WP2)))
    xpt = xp.reshape(nsteps, G, NP2).transpose(0, 2, 1)

    eye16 = jnp.eye(16, dtype=f32)
    w1g = (eye16[None, :, :, None] * w1[:, None, None, :]).reshape(9, 16, 128)
    eye8 = jnp.eye(8, dtype=f32)
    w2g = jax.vmap(lambda m: jnp.kron(eye8, m))(w2.reshape(9, 8, 16))
    eye4 = jnp.eye(4, dtype=f32)
    w3g = jax.vmap(lambda m: jnp.kron(eye4, m))(w3.reshape(9, 16, 32))
    eye2 = jnp.eye(2, dtype=f32)
    w4g = jax.vmap(lambda m: jnp.kron(eye2, m))(w4.reshape(9, 32, 64))
    s1g, t1g = jnp.tile(s1, (1, 16)), jnp.tile(t1, (1, 16))
    s2g, t2g = jnp.tile(s2, (1, 8)), jnp.tile(t2, (1, 8))
    s3g, t3g = jnp.tile(s3, (1, 4)), jnp.tile(t3, (1, 4))
    s4g, t4g = jnp.tile(s4, (1, 2)), jnp.tile(t4, (1, 2))
    se2, so2 = _evenodd(20, WP2)
    se3, so3 = _evenodd(10, WP3)
    se4, so4 = _evenodd(5, WP4)
    ridx = jnp.arange(400)
    perm = jnp.zeros((400, 400), f32).at[
        ridx, (ridx % 16) * 25 + ridx // 16].set(1.0)
    b1_16 = jnp.broadcast_to(f1_b, (16, 256))

    full = lambda shape: pl.BlockSpec(shape, lambda b: tuple(0 for _ in shape))

    grid_spec = pltpu.PrefetchScalarGridSpec(
        num_scalar_prefetch=0,
        grid=(nsteps,),
        in_specs=[
            pl.BlockSpec((1, NP2, G), lambda b: (b, 0, 0)),
            full((9, 16, 128)), full((1, 128)), full((1, 128)),
            full((9, 64, 128)), full((1, 128)), full((1, 128)),
            full((9, 64, 128)), full((1, 128)), full((1, 128)),
            full((9, 64, 128)), full((1, 128)), full((1, 128)),
            full((20, WP2)), full((20, WP2)),
            full((10, WP3)), full((10, WP3)),
            full((5, WP4)), full((5, WP4)),
            full((400, 400)),
            full((1600, 256)), full((16, 256)),
            full((256, 16)), full((1, 16)),
            full((16, 2)), full((1, 2)),
        ],
        out_specs=pl.BlockSpec((G, 2), lambda b: (b, 0)),
        scratch_shapes=[
            pltpu.VMEM((NP2, G * 8), f32),
            pltpu.VMEM((M1, G * 16), f32),
            pltpu.VMEM((NP3, G * 16), f32),
            pltpu.VMEM((M3, G * 32), f32),
            pltpu.VMEM((NP4, G * 32), f32),
            pltpu.VMEM((M4, G * 64), f32),
            pltpu.VMEM((25 * G, 64), f32),
            pltpu.VMEM((400, 64), f32),
        ],
    )
    out = pl.pallas_call(
        _net_kernel,
        out_shape=jax.ShapeDtypeStruct((B, 2), f32),
        grid_spec=grid_spec,
        compiler_params=pltpu.CompilerParams(
            dimension_semantics=("parallel",)),
    )(xpt,
      w1g, s1g, t1g, w2g, s2g, t2g, w3g, s3g, t3g, w4g, s4g, t4g,
      se2, so2, se3, so3, se4, so4, perm,
      f1_w, b1_16, f2_w, f2_b, f3_w, f3_b)
    return out
